# baseline (speedup 1.0000x reference)
import jax
import jax.numpy as jnp
from jax import lax
from jax.experimental import pallas as pl
from jax.experimental.pallas import tpu as pltpu
from jax.experimental.pallas import tpu_sc as plsc

N = 4096
C = 1000
K = int((1.0 - 0.2) * N)

NT = 2560
NSC = N - NT
B = 512
NBT = NT // B
S = 2
BS = B // S

NC = 2
NS = 16
L = 16
NW = NC * NS
ROWS_PER_W = NSC // NW
CH = 16
NCHUNK = ROWS_PER_W // CH
NFULL = C // L - 1
TAIL = C - L

_INTERPRET = False


def _tc_losses_body(*refs):
    lrefs = refs[:2 * S]
    labrefs = refs[2 * S:3 * S]
    out_ref = refs[3 * S]
    for s in range(S):
        lab = labrefs[s][...]
        d1 = lrefs[s][0] - lab
        d2 = lrefs[S + s][0] - lab
        l1 = jnp.sum(d1 * d1, axis=1) * (1.0 / C)
        l2 = jnp.sum(d2 * d2, axis=1) * (1.0 / C)
        out_ref[0, pl.ds(s * BS, BS)] = l1
        out_ref[1, pl.ds(s * BS, BS)] = l2


def _tc_losses(logits, labels):
    in_specs = []
    for m in range(2):
        for s in range(S):
            in_specs.append(
                pl.BlockSpec((1, BS, C), lambda i, m=m, s=s: (m, S * i + s, 0)))
    for s in range(S):
        in_specs.append(pl.BlockSpec((BS, C), lambda i, s=s: (S * i + s, 0)))
    return pl.pallas_call(
        _tc_losses_body,
        grid=(NBT,),
        in_specs=in_specs,
        out_specs=pl.BlockSpec((2, B), lambda i: (0, i)),
        out_shape=jax.ShapeDtypeStruct((2, NT), jnp.float32),
        interpret=_INTERPRET,
    )(*([logits] * (2 * S)), *([labels] * S))


def _sc_losses_body(logits_hbm, labels_hbm, out1_hbm, out2_hbm, b1, b2, bl,
                    o1, o2, sem0, sem1):
    wid = lax.axis_index("s") * NC + lax.axis_index("c")
    base = NT + wid * ROWS_PER_W
    lane = lax.iota(jnp.int32, L)
    sems = (sem0, sem1)

    def start_chunk(ci):
        slot = ci % 2
        r0 = base + ci * CH
        sem = sems[slot]
        return (
            pltpu.async_copy(logits_hbm.at[0, pl.ds(r0, CH), :], b1.at[slot], sem),
            pltpu.async_copy(logits_hbm.at[1, pl.ds(r0, CH), :], b2.at[slot], sem),
            pltpu.async_copy(labels_hbm.at[pl.ds(r0, CH), :], bl.at[slot], sem),
        )

    pending = {0: start_chunk(0)}
    for ci in range(NCHUNK):
        slot = ci % 2
        if ci + 1 < NCHUNK:
            pending[ci + 1] = start_chunk(ci + 1)
        for h in pending.pop(ci):
            h.wait()
        cb1, cb2, cbl = b1.at[slot], b2.at[slot], bl.at[slot]

        def sample_body(s, carry, cb1=cb1, cb2=cb2, cbl=cbl):
            o1v, o2v = carry
            a1 = jnp.zeros((L,), jnp.float32)
            a2 = jnp.zeros((L,), jnp.float32)
            for j in range(NFULL + 1):
                xl = cbl[s, pl.ds(j * L, L)]
                d1 = cb1[s, pl.ds(j * L, L)] - xl
                d2 = cb2[s, pl.ds(j * L, L)] - xl
                a1 = a1 + d1 * d1
                a2 = a2 + d2 * d2
            xl = cbl[s, pl.ds(TAIL, L)]
            d1 = cb1[s, pl.ds(TAIL, L)] - xl
            d2 = cb2[s, pl.ds(TAIL, L)] - xl
            keep = lane >= (L - C % L)
            a1 = a1 + jnp.where(keep, d1 * d1, 0.0)
            a2 = a2 + jnp.where(keep, d2 * d2, 0.0)
            l1 = jnp.sum(a1) * (1.0 / C)
            l2 = jnp.sum(a2) * (1.0 / C)
            ins = lane == s
            return jnp.where(ins, l1, o1v), jnp.where(ins, l2, o2v)

        z16 = jnp.zeros((L,), jnp.float32)
        o1v, o2v = lax.fori_loop(0, CH, sample_body, (z16, z16))
        o1[pl.ds(ci * CH, CH)] = o1v
        o2[pl.ds(ci * CH, CH)] = o2v

    pltpu.sync_copy(o1, out1_hbm.at[pl.ds(wid * ROWS_PER_W, ROWS_PER_W)])
    pltpu.sync_copy(o2, out2_hbm.at[pl.ds(wid * ROWS_PER_W, ROWS_PER_W)])


def _sc_losses(logits, labels):
    mesh = plsc.VectorSubcoreMesh(core_axis_name="c", subcore_axis_name="s")
    f = pl.kernel(
        _sc_losses_body,
        out_type=(jax.ShapeDtypeStruct((NSC,), jnp.float32),
                  jax.ShapeDtypeStruct((NSC,), jnp.float32)),
        mesh=mesh,
        scratch_types=[
            pltpu.VMEM((2, CH, C), jnp.float32),
            pltpu.VMEM((2, CH, C), jnp.float32),
            pltpu.VMEM((2, CH, C), jnp.float32),
            pltpu.VMEM((ROWS_PER_W,), jnp.float32),
            pltpu.VMEM((ROWS_PER_W,), jnp.float32),
            pltpu.SemaphoreType.DMA,
            pltpu.SemaphoreType.DMA,
        ],
        compiler_params=pltpu.CompilerParams(needs_layout_passes=False),
    )
    return f(logits, labels)


def _counts(pieces, thresh1, thresh2):
    c1 = 0
    c2 = 0
    for b1, b2, _ in pieces:
        c1 = c1 + jnp.sum(jnp.where(b1 <= thresh1, 1, 0))
        c2 = c2 + jnp.sum(jnp.where(b2 <= thresh2, 1, 0))
    return c1, c2


def _select_sums(pieces):
    bits = [(lax.bitcast_convert_type(l1, jnp.int32),
             lax.bitcast_convert_type(l2, jnp.int32), idx)
            for l1, l2, idx in pieces]

    def search_val(t, carry):
        lo1, hi1, lo2, hi2 = carry
        m1 = lo1 + (hi1 - lo1) // 2
        m2 = lo2 + (hi2 - lo2) // 2
        c1, c2 = _counts(bits, m1, m2)
        g1 = c1 >= K
        g2 = c2 >= K
        return (jnp.where(g1, lo1, m1 + 1), jnp.where(g1, m1, hi1),
                jnp.where(g2, lo2, m2 + 1), jnp.where(g2, m2, hi2))

    z = jnp.int32(0)
    top = jnp.int32(0x7F800000)
    t1, _, t2, _ = lax.fori_loop(0, 31, search_val, (z, top, z, top))

    cl1, cl2 = _counts(bits, t1 - 1, t2 - 1)
    need1 = K - cl1
    need2 = K - cl2

    def search_idx(t, carry):
        lo1, hi1, lo2, hi2 = carry
        m1 = lo1 + (hi1 - lo1) // 2
        m2 = lo2 + (hi2 - lo2) // 2
        c1 = 0
        c2 = 0
        for b1, b2, idx in bits:
            c1 = c1 + jnp.sum(jnp.where((b1 == t1) & (idx <= m1), 1, 0))
            c2 = c2 + jnp.sum(jnp.where((b2 == t2) & (idx <= m2), 1, 0))
        g1 = c1 >= need1
        g2 = c2 >= need2
        return (jnp.where(g1, lo1, m1 + 1), jnp.where(g1, m1, hi1),
                jnp.where(g2, lo2, m2 + 1), jnp.where(g2, m2, hi2))

    i1, _, i2, _ = lax.fori_loop(0, 12, search_idx,
                                 (z, jnp.int32(N - 1), z, jnp.int32(N - 1)))

    s1 = 0.0
    s2 = 0.0
    for (l1, l2, _), (b1, b2, idx) in zip(pieces, bits):
        mask2 = (b2 < t2) | ((b2 == t2) & (idx <= i2))
        mask1 = (b1 < t1) | ((b1 == t1) & (idx <= i1))
        s1 = s1 + jnp.sum(jnp.where(mask2, l1, 0.0))
        s2 = s2 + jnp.sum(jnp.where(mask1, l2, 0.0))
    return s1, s2


def _select_body(tc_ref, sc1_ref, sc2_ref, out_ref):
    pieces = []
    idx_tc = (lax.broadcasted_iota(jnp.int32, (NBT, B), 0) * B
              + lax.broadcasted_iota(jnp.int32, (NBT, B), 1))
    pieces.append((tc_ref[0], tc_ref[1], idx_tc))
    idx_sc = (NT
              + lax.broadcasted_iota(jnp.int32, (NSC // B, B), 0) * B
              + lax.broadcasted_iota(jnp.int32, (NSC // B, B), 1))
    pieces.append((sc1_ref[...], sc2_ref[...], idx_sc))
    s1, s2 = _select_sums(pieces)
    out_ref[0, 0] = s1 * (1.0 / K)
    out_ref[0, 1] = s2 * (1.0 / K)


def kernel(logits, labels):
    tc_losses = _tc_losses(logits, labels)
    tc_losses = tc_losses.reshape(2, NBT, B)
    sc1, sc2 = _sc_losses(logits, labels)
    sc1 = sc1.reshape(NSC // B, B)
    sc2 = sc2.reshape(NSC // B, B)
    out = pl.pallas_call(
        _select_body,
        out_specs=pl.BlockSpec(memory_space=pltpu.SMEM),
        out_shape=jax.ShapeDtypeStruct((1, 2), jnp.float32),
        interpret=_INTERPRET,
    )(tc_losses, sc1, sc2)
    return (out[0, 0], out[0, 1])
